# baseline (speedup 1.0000x reference)
import functools

import jax
import jax.numpy as jnp
from jax import lax
from jax.experimental import pallas as pl
from jax.experimental.pallas import tpu as pltpu
from jax.experimental.pallas import tpu_sc as plsc

_NUM_CORES = 2
_NUM_SUBCORES = 16
_NW = _NUM_CORES * _NUM_SUBCORES
_BLK = 4096


def _tc_copy(input_logits, memory):
    batch, d = input_logits.shape
    k = memory.shape[0]

    def body(in_ref, mem_ref, out_ref, sem_a, sem_b):
        c0 = pltpu.make_async_copy(
            in_ref, out_ref.at[pl.ds(0, batch)], sem_a
        )
        c1 = pltpu.make_async_copy(
            mem_ref.at[pl.ds(batch, k - batch)],
            out_ref.at[pl.ds(batch, k - batch)],
            sem_b,
        )
        c0.start()
        c1.start()
        c0.wait()
        c1.wait()

    return pl.pallas_call(
        body,
        in_specs=[
            pl.BlockSpec(memory_space=pl.ANY),
            pl.BlockSpec(memory_space=pl.ANY),
        ],
        out_specs=pl.BlockSpec(memory_space=pl.ANY),
        out_shape=jax.ShapeDtypeStruct((k, d), memory.dtype),
        scratch_shapes=[pltpu.SemaphoreType.DMA, pltpu.SemaphoreType.DMA],
    )(input_logits, memory)


def _sc_labels(labels, labels_mem):
    batch = labels.shape[0]
    k = labels_mem.shape[0]
    tail = k - batch

    assert batch % (8 * _NW) == 0 and tail % 8 == 0
    b_per_w = batch // _NW
    t_per_w = -(-(tail // 8) // _NW) * 8

    mesh = plsc.VectorSubcoreMesh(core_axis_name="c", subcore_axis_name="s")

    @functools.partial(
        pl.kernel,
        mesh=mesh,
        out_type=jax.ShapeDtypeStruct((k,), labels_mem.dtype),
        scratch_types=[
            pltpu.VMEM((b_per_w,), labels.dtype),
            pltpu.VMEM((t_per_w,), labels_mem.dtype),
            pltpu.SemaphoreType.DMA,
            pltpu.SemaphoreType.DMA,
        ],
    )
    def sk(lab_hbm, labm_hbm, out_lab, lv, tv, sem_a, sem_b):
        wid = lax.axis_index("s") * _NUM_CORES + lax.axis_index("c")
        ib = wid * b_per_w
        tb = jnp.minimum(batch + wid * t_per_w, k - t_per_w)
        tb = pl.multiple_of(tb, 8)

        c0 = pltpu.async_copy(lab_hbm.at[pl.ds(ib, b_per_w)], lv, sem_a)
        c1 = pltpu.async_copy(labm_hbm.at[pl.ds(tb, t_per_w)], tv, sem_b)
        c0.wait()
        c2 = pltpu.async_copy(lv, out_lab.at[pl.ds(ib, b_per_w)], sem_a)
        c1.wait()
        c3 = pltpu.async_copy(tv, out_lab.at[pl.ds(tb, t_per_w)], sem_b)
        c2.wait()
        c3.wait()

    return sk(labels, labels_mem)


def kernel(input_logits, labels, memory, labels_mem):
    new_memory = _tc_copy(input_logits, memory)
    new_labels_mem = _sc_labels(labels, labels_mem)
    k = memory.shape[0]
    batch = input_logits.shape[0]
    return (new_memory, new_labels_mem, jnp.array(batch % k, dtype=jnp.int32))


# speedup vs baseline: 28.9315x; 28.9315x over previous
import functools

import jax
import jax.numpy as jnp
from jax import lax
from jax.experimental import pallas as pl
from jax.experimental.pallas import tpu as pltpu
from jax.experimental.pallas import tpu_sc as plsc

_NUM_CORES = 2
_NUM_SUBCORES = 16
_NW = _NUM_CORES * _NUM_SUBCORES
_BLK = 4096


def _tc_copy(input_logits, memory):
    batch, d = input_logits.shape
    k = memory.shape[0]

    assert batch % _BLK == 0
    n_in_blocks = batch // _BLK
    grid = (-(-k // _BLK),)

    def body(in_ref, mem_ref, out_ref):
        i = pl.program_id(0)

        @pl.when(i < n_in_blocks)
        def _():
            out_ref[...] = in_ref[...]

        @pl.when(i >= n_in_blocks)
        def _():
            out_ref[...] = mem_ref[...]

    return pl.pallas_call(
        body,
        grid=grid,
        in_specs=[
            pl.BlockSpec(
                (_BLK, d), lambda i: (jnp.minimum(i, n_in_blocks - 1), 0)
            ),
            pl.BlockSpec((_BLK, d), lambda i: (jnp.maximum(i, n_in_blocks), 0)),
        ],
        out_specs=pl.BlockSpec((_BLK, d), lambda i: (i, 0)),
        out_shape=jax.ShapeDtypeStruct((k, d), memory.dtype),
        compiler_params=pltpu.CompilerParams(
            dimension_semantics=("parallel",),
        ),
    )(input_logits, memory)


def _sc_labels(labels, labels_mem):
    batch = labels.shape[0]
    k = labels_mem.shape[0]
    tail = k - batch

    assert batch % (8 * _NW) == 0 and tail % 8 == 0
    b_per_w = batch // _NW
    t_per_w = -(-(tail // 8) // _NW) * 8

    mesh = plsc.VectorSubcoreMesh(core_axis_name="c", subcore_axis_name="s")

    @functools.partial(
        pl.kernel,
        mesh=mesh,
        out_type=jax.ShapeDtypeStruct((k,), labels_mem.dtype),
        scratch_types=[
            pltpu.VMEM((b_per_w,), labels.dtype),
            pltpu.VMEM((t_per_w,), labels_mem.dtype),
            pltpu.SemaphoreType.DMA,
            pltpu.SemaphoreType.DMA,
        ],
    )
    def sk(lab_hbm, labm_hbm, out_lab, lv, tv, sem_a, sem_b):
        wid = lax.axis_index("s") * _NUM_CORES + lax.axis_index("c")
        ib = wid * b_per_w
        tb = jnp.minimum(batch + wid * t_per_w, k - t_per_w)
        tb = pl.multiple_of(tb, 8)

        c0 = pltpu.async_copy(lab_hbm.at[pl.ds(ib, b_per_w)], lv, sem_a)
        c1 = pltpu.async_copy(labm_hbm.at[pl.ds(tb, t_per_w)], tv, sem_b)
        c0.wait()
        c2 = pltpu.async_copy(lv, out_lab.at[pl.ds(ib, b_per_w)], sem_a)
        c1.wait()
        c3 = pltpu.async_copy(tv, out_lab.at[pl.ds(tb, t_per_w)], sem_b)
        c2.wait()
        c3.wait()

    return sk(labels, labels_mem)


def kernel(input_logits, labels, memory, labels_mem):
    new_memory = _tc_copy(input_logits, memory)
    new_labels_mem = _sc_labels(labels, labels_mem)
    k = memory.shape[0]
    batch = input_logits.shape[0]
    return (new_memory, new_labels_mem, jnp.array(batch % k, dtype=jnp.int32))


# speedup vs baseline: 29.0864x; 1.0054x over previous
import functools

import jax
import jax.numpy as jnp
from jax import lax
from jax.experimental import pallas as pl
from jax.experimental.pallas import tpu as pltpu
from jax.experimental.pallas import tpu_sc as plsc

_NUM_CORES = 2
_NUM_SUBCORES = 16
_NW = _NUM_CORES * _NUM_SUBCORES
_BLK = 4096


def _tc_copy(input_logits, memory):
    batch, d = input_logits.shape
    k = memory.shape[0]

    assert batch % _BLK == 0
    n_in = batch // _BLK
    n_t = -(-(k - batch) // _BLK)
    nbuf = 4

    def body(in_ref, mem_ref, out_ref, *scratch):
        bufs = scratch[:nbuf]
        gsems = scratch[nbuf : 2 * nbuf]
        ssems = scratch[2 * nbuf :]

        chunks = [(in_ref, i * _BLK) for i in range(n_in)]
        for i in range(n_t):
            s = min(batch + i * _BLK, k - _BLK)
            chunks.append((mem_ref, s))
        n = len(chunks)

        g = [None] * n
        s_ = [None] * n

        def issue_gather(j):
            if j - nbuf >= 0:
                s_[j - nbuf].wait()
            src, st = chunks[j]
            b = j % nbuf
            g[j] = pltpu.make_async_copy(
                src.at[pl.ds(st, _BLK)], bufs[b], gsems[b]
            )
            g[j].start()

        pref = 2
        for j in range(min(pref, n)):
            issue_gather(j)
        for i in range(n):
            b = i % nbuf
            g[i].wait()
            s_[i] = pltpu.make_async_copy(
                bufs[b], out_ref.at[pl.ds(chunks[i][1], _BLK)], ssems[b]
            )
            s_[i].start()
            if i + pref < n:
                issue_gather(i + pref)
        for i in range(max(0, n - nbuf), n):
            s_[i].wait()

    return pl.pallas_call(
        body,
        in_specs=[
            pl.BlockSpec(memory_space=pl.ANY),
            pl.BlockSpec(memory_space=pl.ANY),
        ],
        out_specs=pl.BlockSpec(memory_space=pl.ANY),
        out_shape=jax.ShapeDtypeStruct((k, d), memory.dtype),
        scratch_shapes=(
            [pltpu.VMEM((_BLK, d), memory.dtype) for _ in range(nbuf)]
            + [pltpu.SemaphoreType.DMA for _ in range(2 * nbuf)]
        ),
    )(input_logits, memory)


def _sc_labels(labels, labels_mem):
    batch = labels.shape[0]
    k = labels_mem.shape[0]
    tail = k - batch

    assert batch % (8 * _NW) == 0 and tail % 8 == 0
    b_per_w = batch // _NW
    t_per_w = -(-(tail // 8) // _NW) * 8

    mesh = plsc.VectorSubcoreMesh(core_axis_name="c", subcore_axis_name="s")

    @functools.partial(
        pl.kernel,
        mesh=mesh,
        out_type=jax.ShapeDtypeStruct((k,), labels_mem.dtype),
        scratch_types=[
            pltpu.VMEM((b_per_w,), labels.dtype),
            pltpu.VMEM((t_per_w,), labels_mem.dtype),
            pltpu.SemaphoreType.DMA,
            pltpu.SemaphoreType.DMA,
        ],
    )
    def sk(lab_hbm, labm_hbm, out_lab, lv, tv, sem_a, sem_b):
        wid = lax.axis_index("s") * _NUM_CORES + lax.axis_index("c")
        ib = wid * b_per_w
        tb = jnp.minimum(batch + wid * t_per_w, k - t_per_w)
        tb = pl.multiple_of(tb, 8)

        c0 = pltpu.async_copy(lab_hbm.at[pl.ds(ib, b_per_w)], lv, sem_a)
        c1 = pltpu.async_copy(labm_hbm.at[pl.ds(tb, t_per_w)], tv, sem_b)
        c0.wait()
        c2 = pltpu.async_copy(lv, out_lab.at[pl.ds(ib, b_per_w)], sem_a)
        c1.wait()
        c3 = pltpu.async_copy(tv, out_lab.at[pl.ds(tb, t_per_w)], sem_b)
        c2.wait()
        c3.wait()

    return sk(labels, labels_mem)


def kernel(input_logits, labels, memory, labels_mem):
    new_memory = _tc_copy(input_logits, memory)
    new_labels_mem = _sc_labels(labels, labels_mem)
    k = memory.shape[0]
    batch = input_logits.shape[0]
    return (new_memory, new_labels_mem, jnp.array(batch % k, dtype=jnp.int32))


# speedup vs baseline: 30.9958x; 1.0656x over previous
import functools

import jax
import jax.numpy as jnp
from jax import lax
from jax.experimental import pallas as pl
from jax.experimental.pallas import tpu as pltpu
from jax.experimental.pallas import tpu_sc as plsc

_NUM_CORES = 2
_NUM_SUBCORES = 16
_NW = _NUM_CORES * _NUM_SUBCORES
_BLK = 4096


def _tc_copy(input_logits, memory):
    batch, d = input_logits.shape
    k = memory.shape[0]

    assert batch % _BLK == 0
    n_in = batch // _BLK
    n_t = -(-(k - batch) // _BLK)
    nbuf = 8

    def body(in_ref, mem_ref, out_ref, *scratch):
        bufs = scratch[:nbuf]
        gsems = scratch[nbuf : 2 * nbuf]
        ssems = scratch[2 * nbuf :]

        chunks = [(in_ref, i * _BLK) for i in range(n_in)]
        for i in range(n_t):
            s = min(batch + i * _BLK, k - _BLK)
            chunks.append((mem_ref, s))
        n = len(chunks)

        g = [None] * n
        s_ = [None] * n

        def issue_gather(j):
            if j - nbuf >= 0:
                s_[j - nbuf].wait()
            src, st = chunks[j]
            b = j % nbuf
            g[j] = pltpu.make_async_copy(
                src.at[pl.ds(st, _BLK)], bufs[b], gsems[b]
            )
            g[j].start()

        pref = 4
        for j in range(min(pref, n)):
            issue_gather(j)
        for i in range(n):
            b = i % nbuf
            g[i].wait()
            s_[i] = pltpu.make_async_copy(
                bufs[b], out_ref.at[pl.ds(chunks[i][1], _BLK)], ssems[b]
            )
            s_[i].start()
            if i + pref < n:
                issue_gather(i + pref)
        for i in range(max(0, n - nbuf), n):
            s_[i].wait()

    return pl.pallas_call(
        body,
        in_specs=[
            pl.BlockSpec(memory_space=pl.ANY),
            pl.BlockSpec(memory_space=pl.ANY),
        ],
        out_specs=pl.BlockSpec(memory_space=pl.ANY),
        out_shape=jax.ShapeDtypeStruct((k, d), memory.dtype),
        scratch_shapes=(
            [pltpu.VMEM((_BLK, d), memory.dtype) for _ in range(nbuf)]
            + [pltpu.SemaphoreType.DMA for _ in range(2 * nbuf)]
        ),
    )(input_logits, memory)


def _sc_labels(labels, labels_mem):
    batch = labels.shape[0]
    k = labels_mem.shape[0]
    tail = k - batch

    assert batch % (8 * _NW) == 0 and tail % 8 == 0
    b_per_w = batch // _NW
    t_per_w = -(-(tail // 8) // _NW) * 8

    mesh = plsc.VectorSubcoreMesh(core_axis_name="c", subcore_axis_name="s")

    @functools.partial(
        pl.kernel,
        mesh=mesh,
        out_type=jax.ShapeDtypeStruct((k,), labels_mem.dtype),
        scratch_types=[
            pltpu.VMEM((b_per_w,), labels.dtype),
            pltpu.VMEM((t_per_w,), labels_mem.dtype),
            pltpu.SemaphoreType.DMA,
            pltpu.SemaphoreType.DMA,
        ],
    )
    def sk(lab_hbm, labm_hbm, out_lab, lv, tv, sem_a, sem_b):
        wid = lax.axis_index("s") * _NUM_CORES + lax.axis_index("c")
        ib = wid * b_per_w
        tb = jnp.minimum(batch + wid * t_per_w, k - t_per_w)
        tb = pl.multiple_of(tb, 8)

        c0 = pltpu.async_copy(lab_hbm.at[pl.ds(ib, b_per_w)], lv, sem_a)
        c1 = pltpu.async_copy(labm_hbm.at[pl.ds(tb, t_per_w)], tv, sem_b)
        c0.wait()
        c2 = pltpu.async_copy(lv, out_lab.at[pl.ds(ib, b_per_w)], sem_a)
        c1.wait()
        c3 = pltpu.async_copy(tv, out_lab.at[pl.ds(tb, t_per_w)], sem_b)
        c2.wait()
        c3.wait()

    return sk(labels, labels_mem)


def kernel(input_logits, labels, memory, labels_mem):
    new_memory = _tc_copy(input_logits, memory)
    new_labels_mem = _sc_labels(labels, labels_mem)
    k = memory.shape[0]
    batch = input_logits.shape[0]
    return (new_memory, new_labels_mem, jnp.array(batch % k, dtype=jnp.int32))


# speedup vs baseline: 31.2743x; 1.0090x over previous
import functools

import jax
import jax.numpy as jnp
from jax import lax
from jax.experimental import pallas as pl
from jax.experimental.pallas import tpu as pltpu
from jax.experimental.pallas import tpu_sc as plsc

_NUM_CORES = 2
_NUM_SUBCORES = 16
_NW = _NUM_CORES * _NUM_SUBCORES
_BLK = 2048


def _tc_copy(input_logits, memory):
    batch, d = input_logits.shape
    k = memory.shape[0]

    assert batch % _BLK == 0
    n_in = batch // _BLK
    n_t = -(-(k - batch) // _BLK)
    nbuf = 16

    def body(in_ref, mem_ref, out_ref, *scratch):
        bufs = scratch[:nbuf]
        gsems = scratch[nbuf : 2 * nbuf]
        ssems = scratch[2 * nbuf :]

        chunks = [(in_ref, i * _BLK) for i in range(n_in)]
        for i in range(n_t):
            s = min(batch + i * _BLK, k - _BLK)
            chunks.append((mem_ref, s))
        n = len(chunks)

        g = [None] * n
        s_ = [None] * n

        def issue_gather(j):
            if j - nbuf >= 0:
                s_[j - nbuf].wait()
            src, st = chunks[j]
            b = j % nbuf
            g[j] = pltpu.make_async_copy(
                src.at[pl.ds(st, _BLK)], bufs[b], gsems[b]
            )
            g[j].start()

        pref = 8
        for j in range(min(pref, n)):
            issue_gather(j)
        for i in range(n):
            b = i % nbuf
            g[i].wait()
            s_[i] = pltpu.make_async_copy(
                bufs[b], out_ref.at[pl.ds(chunks[i][1], _BLK)], ssems[b]
            )
            s_[i].start()
            if i + pref < n:
                issue_gather(i + pref)
        for i in range(max(0, n - nbuf), n):
            s_[i].wait()

    return pl.pallas_call(
        body,
        in_specs=[
            pl.BlockSpec(memory_space=pl.ANY),
            pl.BlockSpec(memory_space=pl.ANY),
        ],
        out_specs=pl.BlockSpec(memory_space=pl.ANY),
        out_shape=jax.ShapeDtypeStruct((k, d), memory.dtype),
        scratch_shapes=(
            [pltpu.VMEM((_BLK, d), memory.dtype) for _ in range(nbuf)]
            + [pltpu.SemaphoreType.DMA for _ in range(2 * nbuf)]
        ),
    )(input_logits, memory)


def _sc_labels(labels, labels_mem):
    batch = labels.shape[0]
    k = labels_mem.shape[0]
    tail = k - batch

    assert batch % (8 * _NW) == 0 and tail % 8 == 0
    b_per_w = batch // _NW
    t_per_w = -(-(tail // 8) // _NW) * 8

    mesh = plsc.VectorSubcoreMesh(core_axis_name="c", subcore_axis_name="s")

    @functools.partial(
        pl.kernel,
        mesh=mesh,
        out_type=jax.ShapeDtypeStruct((k,), labels_mem.dtype),
        scratch_types=[
            pltpu.VMEM((b_per_w,), labels.dtype),
            pltpu.VMEM((t_per_w,), labels_mem.dtype),
            pltpu.SemaphoreType.DMA,
            pltpu.SemaphoreType.DMA,
        ],
    )
    def sk(lab_hbm, labm_hbm, out_lab, lv, tv, sem_a, sem_b):
        wid = lax.axis_index("s") * _NUM_CORES + lax.axis_index("c")
        ib = wid * b_per_w
        tb = jnp.minimum(batch + wid * t_per_w, k - t_per_w)
        tb = pl.multiple_of(tb, 8)

        c0 = pltpu.async_copy(lab_hbm.at[pl.ds(ib, b_per_w)], lv, sem_a)
        c1 = pltpu.async_copy(labm_hbm.at[pl.ds(tb, t_per_w)], tv, sem_b)
        c0.wait()
        c2 = pltpu.async_copy(lv, out_lab.at[pl.ds(ib, b_per_w)], sem_a)
        c1.wait()
        c3 = pltpu.async_copy(tv, out_lab.at[pl.ds(tb, t_per_w)], sem_b)
        c2.wait()
        c3.wait()

    return sk(labels, labels_mem)


def kernel(input_logits, labels, memory, labels_mem):
    new_memory = _tc_copy(input_logits, memory)
    new_labels_mem = _sc_labels(labels, labels_mem)
    k = memory.shape[0]
    batch = input_logits.shape[0]
    return (new_memory, new_labels_mem, jnp.array(batch % k, dtype=jnp.int32))


# speedup vs baseline: 31.7873x; 1.0164x over previous
import functools

import jax
import jax.numpy as jnp
from jax import lax
from jax.experimental import pallas as pl
from jax.experimental.pallas import tpu as pltpu
from jax.experimental.pallas import tpu_sc as plsc

_NUM_CORES = 2
_NUM_SUBCORES = 16
_NW = _NUM_CORES * _NUM_SUBCORES
_BLK = 2048


def _tc_copy(input_logits, memory):
    batch, d = input_logits.shape
    k = memory.shape[0]

    assert batch % _BLK == 0
    n_in = batch // _BLK
    n_t = -(-(k - batch) // _BLK)
    nbuf = 16

    def body(in_ref, mem_ref, out_ref, *scratch):
        bufs = scratch[:nbuf]
        gsems = scratch[nbuf : 2 * nbuf]
        ssems = scratch[2 * nbuf :]

        chunks = [(in_ref, i * _BLK) for i in range(n_in)]
        for i in range(n_t):
            s = min(batch + i * _BLK, k - _BLK)
            chunks.append((mem_ref, s))
        n = len(chunks)

        g = [None] * n
        s_ = [None] * n

        def issue_gather(j):
            if j - nbuf >= 0:
                s_[j - nbuf].wait()
            src, st = chunks[j]
            b = j % nbuf
            g[j] = pltpu.make_async_copy(
                src.at[pl.ds(st, _BLK)], bufs[b], gsems[b]
            )
            g[j].start()

        pref = 12
        for j in range(min(pref, n)):
            issue_gather(j)
        for i in range(n):
            b = i % nbuf
            g[i].wait()
            s_[i] = pltpu.make_async_copy(
                bufs[b], out_ref.at[pl.ds(chunks[i][1], _BLK)], ssems[b]
            )
            s_[i].start()
            if i + pref < n:
                issue_gather(i + pref)
        for i in range(max(0, n - nbuf), n):
            s_[i].wait()

    return pl.pallas_call(
        body,
        in_specs=[
            pl.BlockSpec(memory_space=pl.ANY),
            pl.BlockSpec(memory_space=pl.ANY),
        ],
        out_specs=pl.BlockSpec(memory_space=pl.ANY),
        out_shape=jax.ShapeDtypeStruct((k, d), memory.dtype),
        scratch_shapes=(
            [pltpu.VMEM((_BLK, d), memory.dtype) for _ in range(nbuf)]
            + [pltpu.SemaphoreType.DMA for _ in range(2 * nbuf)]
        ),
    )(input_logits, memory)


def _sc_labels(labels, labels_mem):
    batch = labels.shape[0]
    k = labels_mem.shape[0]
    tail = k - batch

    assert batch % (8 * _NW) == 0 and tail % 8 == 0
    b_per_w = batch // _NW
    t_per_w = -(-(tail // 8) // _NW) * 8

    mesh = plsc.VectorSubcoreMesh(core_axis_name="c", subcore_axis_name="s")

    @functools.partial(
        pl.kernel,
        mesh=mesh,
        out_type=jax.ShapeDtypeStruct((k,), labels_mem.dtype),
        scratch_types=[
            pltpu.VMEM((b_per_w,), labels.dtype),
            pltpu.VMEM((t_per_w,), labels_mem.dtype),
            pltpu.SemaphoreType.DMA,
            pltpu.SemaphoreType.DMA,
        ],
    )
    def sk(lab_hbm, labm_hbm, out_lab, lv, tv, sem_a, sem_b):
        wid = lax.axis_index("s") * _NUM_CORES + lax.axis_index("c")
        ib = wid * b_per_w
        tb = jnp.minimum(batch + wid * t_per_w, k - t_per_w)
        tb = pl.multiple_of(tb, 8)

        c0 = pltpu.async_copy(lab_hbm.at[pl.ds(ib, b_per_w)], lv, sem_a)
        c1 = pltpu.async_copy(labm_hbm.at[pl.ds(tb, t_per_w)], tv, sem_b)
        c0.wait()
        c2 = pltpu.async_copy(lv, out_lab.at[pl.ds(ib, b_per_w)], sem_a)
        c1.wait()
        c3 = pltpu.async_copy(tv, out_lab.at[pl.ds(tb, t_per_w)], sem_b)
        c2.wait()
        c3.wait()

    return sk(labels, labels_mem)


def kernel(input_logits, labels, memory, labels_mem):
    new_memory = _tc_copy(input_logits, memory)
    new_labels_mem = _sc_labels(labels, labels_mem)
    k = memory.shape[0]
    batch = input_logits.shape[0]
    return (new_memory, new_labels_mem, jnp.array(batch % k, dtype=jnp.int32))
